# baseline (speedup 1.0000x reference)
import jax
import jax.numpy as jnp
from jax import lax
from jax.experimental import pallas as pl
from jax.experimental.pallas import tpu as pltpu, tpu_sc as plsc

_B = 4
_S = 4096
_D = 1024
_NC = 2
_NS = 16
_NW = _NC * _NS
_CHUNK = (_B * _S) // _NW
_CPR = _S // _CHUNK
_VPC = _CHUNK // 16
_K = 16


def _sc_body(ids_hbm, dig_hbm, table_hbm, out_hbm,
             dig_v, idsrow_v, tloc_v, fbuf_v, sem_f, sem_g, sem_t, sem_s):
    c = lax.axis_index("c")
    s = lax.axis_index("s")
    row = c * (_B // _NC) + s // _CPR
    cl = s % _CPR

    tload = pltpu.async_copy(table_hbm.at[pl.ds(0, _K)], tloc_v, sem_t)

    pltpu.sync_copy(dig_hbm, dig_v)
    dv = dig_v[...]
    dmsk = (dv >= 0) & (dv < 32)
    bits = jnp.where(dmsk, jnp.left_shift(jnp.uint32(1), dv.astype(jnp.uint32)),
                     jnp.zeros((16,), jnp.uint32))
    bitmap = jnp.sum(bits)

    iload = pltpu.async_copy(ids_hbm.at[pl.ds(row * _S, _S)], idsrow_v, sem_g)

    iota = lax.iota(jnp.int32, 16)
    neg1 = jnp.full((16,), -1, jnp.int32)

    def tvec(j):
        x = jnp.clip(idsrow_v[pl.ds(j * 16, 16)], 0, 31).astype(jnp.uint32)
        m = (jnp.right_shift(lax.broadcast_in_dim(bitmap, (16,), ()), x)
             & jnp.uint32(1)).astype(jnp.int32)
        gidx = j * 16 + iota
        t = jnp.where(m == 1, neg1, gidx)
        return m, gidx, t

    iload.wait()

    def pref_body(j, carryv):
        for u in range(4):
            _, _, t = tvec(j * 4 + u)
            carryv = jnp.maximum(carryv, t)
        return carryv

    carryv0 = lax.fori_loop(0, cl * (_VPC // 4), pref_body, neg1)
    carry = jnp.max(carryv0)

    tload.wait()

    out_base = row * _S + cl * _CHUNK

    def step(jj, cin):
        carry_s, flagv = cin
        m, gidx, t = tvec(cl * _VPC + jj)
        pm = jnp.maximum(plsc.cummax(t),
                         lax.broadcast_in_dim(carry_s, (16,), ()))
        pos = (gidx - pm) * m
        cnt = plsc.all_reduce_population_count(pos >= _K)
        flagv = flagv | jnp.where(
            cnt > 0,
            jnp.left_shift(jnp.full((16,), 1, jnp.uint32),
                           lax.broadcast_in_dim(jj.astype(jnp.uint32),
                                                (16,), ())),
            jnp.zeros((16,), jnp.uint32))
        need = cnt[0]

        def fast(_):
            for t_ in range(16):
                p = pos[t_]
                pltpu.async_copy(
                    tloc_v.at[pl.ds(p, 1)],
                    out_hbm.at[pl.ds(out_base + jj * 16 + t_, 1)], sem_f)
            return 0

        def slow(_):
            pltpu.async_copy(table_hbm.at[pos], fbuf_v, sem_g).wait()
            for t_ in range(16):
                pltpu.async_copy(
                    fbuf_v.at[pl.ds(t_, 1)],
                    out_hbm.at[pl.ds(out_base + jj * 16 + t_, 1)], sem_s)
            pltpu.make_async_copy(
                table_hbm.at[pl.ds(0, 16)], fbuf_v, sem_s).wait()
            return 0

        lax.cond(need == 0, fast, slow, 0)
        return (pm[15], flagv)

    _, flagv_fin = pl.loop(
        0, _VPC, init_carry=(carry, jnp.zeros((16,), jnp.uint32)))(step)

    iota_u = iota.astype(jnp.uint32)
    nslow = jnp.sum(
        (jnp.right_shift(flagv_fin, iota_u) & jnp.uint32(1)).astype(jnp.int32)
        + (jnp.right_shift(flagv_fin, iota_u + 16)
           & jnp.uint32(1)).astype(jnp.int32))
    nfast = _VPC - nslow

    def drain_body(i, z):
        pltpu.make_async_copy(table_hbm.at[pl.ds(0, 16)], fbuf_v, sem_f).wait()
        return z

    lax.fori_loop(0, nfast, drain_body, 0)


def kernel(input_ids, digits, table):
    ids_flat = input_ids.reshape(_B * _S).astype(jnp.int32)
    dig16 = jnp.concatenate(
        [digits.astype(jnp.int32),
         jnp.full((16 - digits.shape[0],), -1, jnp.int32)])

    mesh = plsc.VectorSubcoreMesh(core_axis_name="c", subcore_axis_name="s")
    run = pl.kernel(
        _sc_body,
        out_type=jax.ShapeDtypeStruct((_B * _S, _D), jnp.float32),
        mesh=mesh,
        compiler_params=pltpu.CompilerParams(needs_layout_passes=False),
        scratch_types=[
            pltpu.VMEM((16,), jnp.int32),
            pltpu.VMEM((_S,), jnp.int32),
            pltpu.VMEM((_K, _D), jnp.float32),
            pltpu.VMEM((16, _D), jnp.float32),
            pltpu.SemaphoreType.DMA,
            pltpu.SemaphoreType.DMA,
            pltpu.SemaphoreType.DMA,
            pltpu.SemaphoreType.DMA,
        ],
    )
    out = run(ids_flat, dig16, table)
    return out.reshape(_B, _S, _D)


# speedup vs baseline: 1.0199x; 1.0199x over previous
import jax
import jax.numpy as jnp
from jax import lax
from jax.experimental import pallas as pl
from jax.experimental.pallas import tpu as pltpu, tpu_sc as plsc

_B = 4
_S = 4096
_D = 1024
_NC = 2
_NS = 16
_NW = _NC * _NS
_CHUNK = (_B * _S) // _NW
_CPR = _S // _CHUNK
_VPC = _CHUNK // 16
_K = 32


def _sc_body(ids_hbm, dig_hbm, table_hbm, out_hbm,
             dig_v, idsrow_v, tloc_v, fbuf_v, sem_f, sem_g, sem_t, sem_s):
    c = lax.axis_index("c")
    s = lax.axis_index("s")
    row = c * (_B // _NC) + s // _CPR
    cl = s % _CPR

    tload = pltpu.async_copy(table_hbm.at[pl.ds(0, _K)], tloc_v, sem_t)

    pltpu.sync_copy(dig_hbm, dig_v)
    dv = dig_v[...]
    dmsk = (dv >= 0) & (dv < 32)
    bits = jnp.where(dmsk, jnp.left_shift(jnp.uint32(1), dv.astype(jnp.uint32)),
                     jnp.zeros((16,), jnp.uint32))
    bitmap = jnp.sum(bits)

    iload = pltpu.async_copy(ids_hbm.at[pl.ds(row * _S, _S)], idsrow_v, sem_g)

    iota = lax.iota(jnp.int32, 16)
    neg1 = jnp.full((16,), -1, jnp.int32)

    def tvec(j):
        x = jnp.clip(idsrow_v[pl.ds(j * 16, 16)], 0, 31).astype(jnp.uint32)
        m = (jnp.right_shift(lax.broadcast_in_dim(bitmap, (16,), ()), x)
             & jnp.uint32(1)).astype(jnp.int32)
        gidx = j * 16 + iota
        t = jnp.where(m == 1, neg1, gidx)
        return m, gidx, t

    iload.wait()

    def pref_body(j, carryv):
        for u in range(4):
            _, _, t = tvec(j * 4 + u)
            carryv = jnp.maximum(carryv, t)
        return carryv

    carryv0 = lax.fori_loop(0, cl * (_VPC // 4), pref_body, neg1)
    carry = jnp.max(carryv0)

    tload.wait()

    out_base = row * _S + cl * _CHUNK

    def step(jj, cin):
        carry_s, flagv = cin
        m, gidx, t = tvec(cl * _VPC + jj)
        pm = jnp.maximum(plsc.cummax(t),
                         lax.broadcast_in_dim(carry_s, (16,), ()))
        pos = (gidx - pm) * m
        cnt = plsc.all_reduce_population_count(pos >= _K)
        flagv = flagv | jnp.where(
            cnt > 0,
            jnp.left_shift(jnp.full((16,), 1, jnp.uint32),
                           lax.broadcast_in_dim(jj.astype(jnp.uint32),
                                                (16,), ())),
            jnp.zeros((16,), jnp.uint32))
        need = cnt[0]

        def fast(_):
            for t_ in range(16):
                p = pos[t_]
                pltpu.async_copy(
                    tloc_v.at[pl.ds(p, 1)],
                    out_hbm.at[pl.ds(out_base + jj * 16 + t_, 1)], sem_f)
            return 0

        def slow(_):
            pltpu.async_copy(table_hbm.at[pos], fbuf_v, sem_g).wait()
            for t_ in range(16):
                pltpu.async_copy(
                    fbuf_v.at[pl.ds(t_, 1)],
                    out_hbm.at[pl.ds(out_base + jj * 16 + t_, 1)], sem_s)
            pltpu.make_async_copy(
                table_hbm.at[pl.ds(0, 16)], fbuf_v, sem_s).wait()
            return 0

        lax.cond(need == 0, fast, slow, 0)
        return (pm[15], flagv)

    _, flagv_fin = pl.loop(
        0, _VPC, init_carry=(carry, jnp.zeros((16,), jnp.uint32)))(step)

    iota_u = iota.astype(jnp.uint32)
    nslow = jnp.sum(
        (jnp.right_shift(flagv_fin, iota_u) & jnp.uint32(1)).astype(jnp.int32)
        + (jnp.right_shift(flagv_fin, iota_u + 16)
           & jnp.uint32(1)).astype(jnp.int32))
    nfast = _VPC - nslow

    def drain_body(i, z):
        pltpu.make_async_copy(table_hbm.at[pl.ds(0, 16)], fbuf_v, sem_f).wait()
        return z

    lax.fori_loop(0, nfast, drain_body, 0)


def kernel(input_ids, digits, table):
    ids_flat = input_ids.reshape(_B * _S).astype(jnp.int32)
    dig16 = jnp.concatenate(
        [digits.astype(jnp.int32),
         jnp.full((16 - digits.shape[0],), -1, jnp.int32)])

    mesh = plsc.VectorSubcoreMesh(core_axis_name="c", subcore_axis_name="s")
    run = pl.kernel(
        _sc_body,
        out_type=jax.ShapeDtypeStruct((_B * _S, _D), jnp.float32),
        mesh=mesh,
        compiler_params=pltpu.CompilerParams(needs_layout_passes=False),
        scratch_types=[
            pltpu.VMEM((16,), jnp.int32),
            pltpu.VMEM((_S,), jnp.int32),
            pltpu.VMEM((_K, _D), jnp.float32),
            pltpu.VMEM((16, _D), jnp.float32),
            pltpu.SemaphoreType.DMA,
            pltpu.SemaphoreType.DMA,
            pltpu.SemaphoreType.DMA,
            pltpu.SemaphoreType.DMA,
        ],
    )
    out = run(ids_flat, dig16, table)
    return out.reshape(_B, _S, _D)


# speedup vs baseline: 1.0340x; 1.0138x over previous
import jax
import jax.numpy as jnp
from jax import lax
from jax.experimental import pallas as pl
from jax.experimental.pallas import tpu as pltpu, tpu_sc as plsc

_B = 4
_S = 4096
_D = 1024
_NC = 2
_NS = 16
_NW = _NC * _NS
_CHUNK = (_B * _S) // _NW
_CPR = _S // _CHUNK
_VPC = _CHUNK // 16
_K = 32


def _sc_body(ids_hbm, dig_hbm, table_hbm, out_hbm,
             dig_v, idsrow_v, tloc_v, fbuf_v, sem_f, sem_g, sem_t, sem_s):
    c = lax.axis_index("c")
    s = lax.axis_index("s")
    row = c * (_B // _NC) + s // _CPR
    cl = s % _CPR

    tload = pltpu.async_copy(table_hbm.at[pl.ds(0, _K)], tloc_v, sem_t)

    pltpu.sync_copy(dig_hbm, dig_v)
    dv = dig_v[...]
    dmsk = (dv >= 0) & (dv < 32)
    bits = jnp.where(dmsk, jnp.left_shift(jnp.uint32(1), dv.astype(jnp.uint32)),
                     jnp.zeros((16,), jnp.uint32))
    bitmap = jnp.sum(bits)

    iload = pltpu.async_copy(ids_hbm.at[pl.ds(row * _S, _S)], idsrow_v, sem_g)

    iota = lax.iota(jnp.int32, 16)
    neg1 = jnp.full((16,), -1, jnp.int32)

    def tvec(j):
        x = jnp.clip(idsrow_v[pl.ds(j * 16, 16)], 0, 31).astype(jnp.uint32)
        m = (jnp.right_shift(lax.broadcast_in_dim(bitmap, (16,), ()), x)
             & jnp.uint32(1)).astype(jnp.int32)
        gidx = j * 16 + iota
        t = jnp.where(m == 1, neg1, gidx)
        return m, gidx, t

    iload.wait()

    def pref_body(j, carryv):
        for u in range(4):
            _, _, t = tvec(j * 4 + u)
            carryv = jnp.maximum(carryv, t)
        return carryv

    carryv0 = lax.fori_loop(0, cl * (_VPC // 4), pref_body, neg1)
    carry = jnp.max(carryv0)

    tload.wait()

    out_base = row * _S + cl * _CHUNK

    def step(jj, cin):
        carry_s, flagv = cin
        m, gidx, t = tvec(cl * _VPC + jj)
        pm = jnp.maximum(plsc.cummax(t),
                         lax.broadcast_in_dim(carry_s, (16,), ()))
        pos = (gidx - pm) * m
        cnt = plsc.all_reduce_population_count(pos >= _K)
        flagv = flagv | jnp.where(
            cnt > 0,
            jnp.left_shift(jnp.full((16,), 1, jnp.uint32),
                           lax.broadcast_in_dim(jj.astype(jnp.uint32),
                                                (16,), ())),
            jnp.zeros((16,), jnp.uint32))
        need = cnt[0]

        def fast(_):
            for t_ in range(16):
                p = pos[t_]
                pltpu.async_copy(
                    tloc_v.at[pl.ds(p, 1)],
                    out_hbm.at[pl.ds(out_base + jj * 16 + t_, 1)], sem_f)
            return 0

        def slow(_):
            pltpu.async_copy(table_hbm.at[pos], fbuf_v, sem_g).wait()
            for t_ in range(16):
                pltpu.async_copy(
                    fbuf_v.at[pl.ds(t_, 1)],
                    out_hbm.at[pl.ds(out_base + jj * 16 + t_, 1)], sem_s)
            pltpu.make_async_copy(
                table_hbm.at[pl.ds(0, 16)], fbuf_v, sem_s).wait()
            return 0

        lax.cond(need == 0, fast, slow, 0)
        return (pm[15], flagv)

    _, flagv_fin = pl.loop(
        0, _VPC, init_carry=(carry, jnp.zeros((16,), jnp.uint32)))(step)

    iota_u = iota.astype(jnp.uint32)
    nslow = jnp.sum(
        (jnp.right_shift(flagv_fin, iota_u) & jnp.uint32(1)).astype(jnp.int32)
        + (jnp.right_shift(flagv_fin, iota_u + 16)
           & jnp.uint32(1)).astype(jnp.int32))
    nfast = _VPC - nslow

    def drain_body(i, z):
        pltpu.make_async_copy(table_hbm.at[pl.ds(0, 16)], fbuf_v, sem_f).wait()
        return z

    lax.fori_loop(0, nfast, drain_body, 0)


def kernel(input_ids, digits, table):
    ids_flat = input_ids.reshape(_B * _S).astype(jnp.int32)
    dig16 = jnp.concatenate(
        [digits.astype(jnp.int32),
         jnp.full((16 - digits.shape[0],), -1, jnp.int32)])

    mesh = plsc.VectorSubcoreMesh(core_axis_name="c", subcore_axis_name="s",
                                  num_cores=_NC, num_subcores=_NS)
    run = pl.kernel(
        _sc_body,
        out_type=jax.ShapeDtypeStruct((_B * _S, _D), jnp.float32),
        mesh=mesh,
        compiler_params=pltpu.CompilerParams(needs_layout_passes=False),
        scratch_types=[
            pltpu.VMEM((16,), jnp.int32),
            pltpu.VMEM((_S,), jnp.int32),
            pltpu.VMEM((_K, _D), jnp.float32),
            pltpu.VMEM((16, _D), jnp.float32),
            pltpu.SemaphoreType.DMA,
            pltpu.SemaphoreType.DMA,
            pltpu.SemaphoreType.DMA,
            pltpu.SemaphoreType.DMA,
        ],
    )
    out = run(ids_flat, dig16, table)
    return out.reshape(_B, _S, _D)
